# baseline (speedup 1.0000x reference)
import functools

import jax
import jax.numpy as jnp
from jax import lax
from jax.experimental import pallas as pl
from jax.experimental.pallas import tpu as pltpu
from jax.experimental.pallas import tpu_sc as plsc

N = 10000
K = 32
D = 128
OUT = 128

NC = 2
NS = 16
LANES = 16

B = 8
BK = B * K
G = N // B
GC = G // NC
GW = GC // NS
MAXB = GW + 1
ROWS_W = GW * B
ZROWS = 104


def _sc_aggregate(h_nn2d):
    mesh = plsc.VectorSubcoreMesh(core_axis_name="c", subcore_axis_name="s")

    @functools.partial(
        pl.kernel,
        mesh=mesh,
        out_type=jax.ShapeDtypeStruct((N, D), jnp.float32),
        scratch_types=[
            pltpu.VMEM((BK, D), jnp.float32),
            pltpu.VMEM((BK, D), jnp.float32),
            pltpu.VMEM((BK,), jnp.int32),
            pltpu.VMEM((BK,), jnp.int32),
            pltpu.VMEM((ZROWS, D), jnp.float32),
            pltpu.VMEM_SHARED((GC * B, D), jnp.float32),
            pltpu.SemaphoreType.DMA,
            pltpu.SemaphoreType.DMA,
            pltpu.SemaphoreType.DMA,
            pltpu.SemaphoreType.DMA,
        ],
    )
    def aggr_kernel(hnn_hbm, out_hbm, buf0, buf1, idx0, idx1, zbuf, acc,
                    si0, si1, ss0, ss1):
        c = lax.axis_index("c")
        s = lax.axis_index("s")
        start_g = c * GC + s * GW
        rel_row = s * ROWS_W

        def start_in(i, buf, sem):
            g = start_g + i
            pltpu.async_copy(hnn_hbm.at[pl.ds(g * BK, BK)], buf, sem)

        def wait_in(buf, sem):
            pltpu.make_async_copy(hnn_hbm.at[pl.ds(0, BK)], buf, sem).wait()

        def scatter(buf, idx, sem):
            pltpu.async_copy(buf, acc.at[idx], sem, add=True)

        def wait_sc(buf, idx, sem):
            pltpu.make_async_copy(buf, acc.at[idx], sem).wait()

        def fill_idx(idx, i):
            node0 = rel_row + i * B
            for v in range(BK // LANES):
                idx[pl.ds(v * LANES, LANES)] = jnp.full(
                    (LANES,), node0 + v // 2, jnp.int32)

        for r in range(ZROWS):
            for cb in range(D // LANES):
                zbuf[r, pl.ds(cb * LANES, LANES)] = jnp.zeros(
                    (LANES,), jnp.float32)
        for z in range(ROWS_W // ZROWS):
            pltpu.sync_copy(zbuf, acc.at[pl.ds(rel_row + z * ZROWS, ZROWS)])

        @pl.when(s == NS - 1)
        def _zero_tail():
            pltpu.sync_copy(zbuf.at[pl.ds(0, B)],
                            acc.at[pl.ds(rel_row + ROWS_W, B)])

        fill_idx(idx0, 0)
        fill_idx(idx1, 1)
        start_in(0, buf0, si0)
        start_in(1, buf1, si1)

        def body(t, carry):
            i0 = 2 * t
            wait_in(buf0, si0)
            scatter(buf0, idx0, ss0)
            wait_in(buf1, si1)
            scatter(buf1, idx1, ss1)
            wait_sc(buf0, idx0, ss0)
            fill_idx(idx0, i0 + 2)
            start_in(i0 + 2, buf0, si0)
            wait_sc(buf1, idx1, ss1)
            fill_idx(idx1, i0 + 3)
            start_in(i0 + 3, buf1, si1)
            return carry

        lax.fori_loop(0, (GW - 3) // 2, body, 0, unroll=False)

        wait_in(buf0, si0)
        scatter(buf0, idx0, ss0)
        wait_in(buf1, si1)
        scatter(buf1, idx1, ss1)
        wait_sc(buf0, idx0, ss0)
        fill_idx(idx0, GW - 1)
        start_in(GW - 1, buf0, si0)
        wait_in(buf0, si0)
        scatter(buf0, idx0, ss0)

        @pl.when(s == NS - 1)
        def _tail_group():
            wait_sc(buf1, idx1, ss1)
            fill_idx(idx1, GW)
            start_in(GW, buf1, si1)
            wait_in(buf1, si1)
            scatter(buf1, idx1, ss1)

        wait_sc(buf0, idx0, ss0)
        wait_sc(buf1, idx1, ss1)

        out0 = c * GC * B + rel_row
        pltpu.sync_copy(acc.at[pl.ds(rel_row, ROWS_W)],
                        out_hbm.at[pl.ds(out0, ROWS_W)])

        @pl.when(s == NS - 1)
        def _out_tail():
            pltpu.sync_copy(acc.at[pl.ds(rel_row + ROWS_W, B)],
                            out_hbm.at[pl.ds(out0 + ROWS_W, B)])

    return aggr_kernel(h_nn2d)


BLOCK_M = 1000


def _tc_body(h_ref, aggr_ref, w1_ref, w2_ref, b_ref, o_ref):
    o_ref[...] = (
        jnp.dot(h_ref[...], w1_ref[...], preferred_element_type=jnp.float32)
        + jnp.dot(aggr_ref[...], w2_ref[...], preferred_element_type=jnp.float32)
        + b_ref[...]
    )


def _tc_linear(h, aggr, W, b):
    w1t = W[:, :D].T
    w2t = W[:, D:].T
    b2 = b.reshape(1, OUT)
    return pl.pallas_call(
        _tc_body,
        grid=(N // BLOCK_M,),
        in_specs=[
            pl.BlockSpec((BLOCK_M, D), lambda i: (i, 0)),
            pl.BlockSpec((BLOCK_M, D), lambda i: (i, 0)),
            pl.BlockSpec((D, OUT), lambda i: (0, 0)),
            pl.BlockSpec((D, OUT), lambda i: (0, 0)),
            pl.BlockSpec((1, OUT), lambda i: (0, 0)),
        ],
        out_specs=pl.BlockSpec((BLOCK_M, OUT), lambda i: (i, 0)),
        out_shape=jax.ShapeDtypeStruct((N, OUT), jnp.float32),
    )(h, aggr, w1t, w2t, b2)


def kernel(h, h_nn, W, b):
    aggr = _sc_aggregate(h_nn.reshape(N * K, D))
    return _tc_linear(h, aggr, W, b)


# speedup vs baseline: 2.8740x; 2.8740x over previous
import jax
import jax.numpy as jnp
from jax.experimental import pallas as pl

N = 10000
K = 32
D = 128
OUT = 128
BLOCK_M = 400


def _body(h_ref, hnn_a_ref, hnn_b_ref, w1_ref, w2_ref, b_ref, o_ref):
    aggr = jnp.sum(hnn_a_ref[...], axis=1) + jnp.sum(hnn_b_ref[...], axis=1)
    o_ref[...] = (
        jnp.dot(h_ref[...], w1_ref[...], preferred_element_type=jnp.float32)
        + jnp.dot(aggr, w2_ref[...], preferred_element_type=jnp.float32)
        + b_ref[...]
    )


def kernel(h, h_nn, W, b):
    w1t = W[:, :D].T
    w2t = W[:, D:].T
    b2 = b.reshape(1, OUT)
    grid = (N // BLOCK_M,)
    return pl.pallas_call(
        _body,
        grid=grid,
        in_specs=[
            pl.BlockSpec((BLOCK_M, D), lambda i: (i, 0)),
            pl.BlockSpec((BLOCK_M, K // 2, D), lambda i: (i, 0, 0)),
            pl.BlockSpec((BLOCK_M, K // 2, D), lambda i: (i, 1, 0)),
            pl.BlockSpec((D, OUT), lambda i: (0, 0)),
            pl.BlockSpec((D, OUT), lambda i: (0, 0)),
            pl.BlockSpec((1, OUT), lambda i: (0, 0)),
        ],
        out_specs=pl.BlockSpec((BLOCK_M, OUT), lambda i: (i, 0)),
        out_shape=jax.ShapeDtypeStruct((N, OUT), jnp.float32),
    )(h, h_nn, h_nn, w1t, w2t, b2)


# speedup vs baseline: 2.9280x; 1.0188x over previous
import jax
import jax.numpy as jnp
from jax.experimental import pallas as pl

N = 10000
K = 32
D = 128
OUT = 128
BLOCK_M = 400


def _body(h_ref, hnn_ref, w1_ref, w2_ref, b_ref, o_ref):
    aggr = jnp.sum(hnn_ref[...], axis=1)
    o_ref[...] = (
        jnp.dot(h_ref[...], w1_ref[...], preferred_element_type=jnp.float32)
        + jnp.dot(aggr, w2_ref[...], preferred_element_type=jnp.float32)
        + b_ref[...]
    )


def kernel(h, h_nn, W, b):
    w1t = W[:, :D].T
    w2t = W[:, D:].T
    b2 = b.reshape(1, OUT)
    grid = (N // BLOCK_M,)
    return pl.pallas_call(
        _body,
        grid=grid,
        in_specs=[
            pl.BlockSpec((BLOCK_M, D), lambda i: (i, 0)),
            pl.BlockSpec((BLOCK_M, K, D), lambda i: (i, 0, 0)),
            pl.BlockSpec((D, OUT), lambda i: (0, 0)),
            pl.BlockSpec((D, OUT), lambda i: (0, 0)),
            pl.BlockSpec((1, OUT), lambda i: (0, 0)),
        ],
        out_specs=pl.BlockSpec((BLOCK_M, OUT), lambda i: (i, 0)),
        out_shape=jax.ShapeDtypeStruct((N, OUT), jnp.float32),
    )(h, h_nn, w1t, w2t, b2)
